# Initial kernel scaffold; baseline (speedup 1.0000x reference)
import jax
import jax.numpy as jnp
from jax.experimental import pallas as pl


def kernel(x, edge_index, w1, b1, w2, b2, ln_gamma, ln_beta):
    raise NotImplementedError("write your pallas kernel here")


# speedup vs baseline: 1.0006x; 1.0006x over previous
import functools

import jax
import jax.numpy as jnp
from jax.experimental import pallas as pl

N = 10000
E = 320000
D = 128
ALPHA = 0.1
HOPS = 10


def _ffn_body(h_ref, x_ref, w1_ref, b1_ref, w2_ref, b2_ref, g_ref, bt_ref, o_ref):
    h = h_ref[...]
    mu = jnp.mean(h, axis=1, keepdims=True)
    var = jnp.mean((h - mu) ** 2, axis=1, keepdims=True)
    hn = (h - mu) * jax.lax.rsqrt(var + 1e-5) * g_ref[...] + bt_ref[...]
    inter = jnp.maximum(
        jnp.dot(hn, w1_ref[...], preferred_element_type=jnp.float32) + b1_ref[...], 0.0
    )
    ff = jnp.dot(inter, w2_ref[...], preferred_element_type=jnp.float32) + b2_ref[...]
    o_ref[...] = ff + x_ref[...]


def _ffn(h, x, w1, b1, w2, b2, g, bt):
    R = 2000
    grid = (N // R,)
    row_spec = pl.BlockSpec((R, D), lambda i: (i, 0))
    full_spec = pl.BlockSpec((D, D), lambda i: (0, 0))
    vec_spec = pl.BlockSpec((1, D), lambda i: (0, 0))
    return pl.pallas_call(
        _ffn_body,
        grid=grid,
        in_specs=[row_spec, row_spec, full_spec, vec_spec, full_spec, vec_spec,
                  vec_spec, vec_spec],
        out_specs=row_spec,
        out_shape=jax.ShapeDtypeStruct((N, D), jnp.float32),
    )(h, x, w1, b1.reshape(1, D), w2, b2.reshape(1, D), g.reshape(1, D),
      bt.reshape(1, D))


def kernel(x, edge_index, w1, b1, w2, b2, ln_gamma, ln_beta):
    src = edge_index[0]
    dst = edge_index[1]
    deg = jax.ops.segment_sum(jnp.ones((E,), dtype=jnp.float32), dst, num_segments=N)
    deg = jnp.where(deg > 0, deg, 1.0)
    norm = (deg ** -0.5)[:, None]
    feat0 = x
    h = x
    for _ in range(HOPS):
        h = h * norm
        msg = jnp.take(h, src, axis=0)
        agg = jax.ops.segment_sum(msg, dst, num_segments=N)
        h = agg * norm
        h = (1.0 - ALPHA) * h + ALPHA * feat0
    r = h
    rst = _ffn(h, x, w1, b1, w2, b2, ln_gamma, ln_beta)
    return (rst, r)


# speedup vs baseline: 7.1567x; 7.1523x over previous
import functools

import jax
import jax.numpy as jnp
from jax import lax
from jax.experimental import pallas as pl
from jax.experimental.pallas import tpu as pltpu
from jax.experimental.pallas import tpu_sc as plsc

N = 10000
E = 320000
D = 128
ALPHA = 0.1
HOPS = 10

NC = 2
NS = 16
NW = NC * NS
NPAD = 10240
RPW = NPAD // NS
EPW = E // NW
B = 80
NB = EPW // B

_MESH = plsc.VectorSubcoreMesh(
    core_axis_name="c", subcore_axis_name="s", num_cores=NC, num_subcores=NS
)


def _deg_body(dst3, z1, out, dst_v, ones_v, deg_sh):
    c = lax.axis_index("c")
    s = lax.axis_index("s")
    w = c * NS + s
    for j in range(B // 16):
        ones_v[pl.ds(j * 16, 16)] = jnp.full((16,), 1.0, jnp.float32)
    pltpu.sync_copy(z1.at[pl.ds(s * RPW, RPW)], deg_sh.at[pl.ds(s * RPW, RPW)])
    plsc.subcore_barrier()
    pltpu.sync_copy(dst3.at[w], dst_v)

    def body(i, carry):
        pltpu.sync_copy(ones_v, deg_sh.at[dst_v.at[i]], add=True)
        return carry

    lax.fori_loop(0, NB, body, 0)
    plsc.subcore_barrier()
    pltpu.sync_copy(deg_sh.at[pl.ds(s * RPW, RPW)], out.at[c, pl.ds(s * RPW, RPW)])


_deg_call = pl.kernel(
    _deg_body,
    out_type=jax.ShapeDtypeStruct((NC, NPAD), jnp.float32),
    mesh=_MESH,
    scratch_types=[
        pltpu.VMEM((NB, B), jnp.int32),
        pltpu.VMEM((B,), jnp.float32),
        pltpu.VMEM_SHARED((NPAD,), jnp.float32),
    ],
)


def _hop_body(hs, src3, dst3, z2, out, src_v, dst_v, rows_v, agg_sh):
    c = lax.axis_index("c")
    s = lax.axis_index("s")
    w = c * NS + s
    pltpu.sync_copy(z2.at[pl.ds(s * RPW, RPW)], agg_sh.at[pl.ds(s * RPW, RPW)])
    plsc.subcore_barrier()
    pltpu.sync_copy(src3.at[w], src_v)
    pltpu.sync_copy(dst3.at[w], dst_v)

    def body(i, carry):
        pltpu.sync_copy(hs.at[src_v.at[i]], rows_v)
        pltpu.sync_copy(rows_v, agg_sh.at[dst_v.at[i]], add=True)
        return carry

    lax.fori_loop(0, NB, body, 0)
    plsc.subcore_barrier()
    pltpu.sync_copy(agg_sh.at[pl.ds(s * RPW, RPW)], out.at[c, pl.ds(s * RPW, RPW)])


_hop_call = pl.kernel(
    _hop_body,
    out_type=jax.ShapeDtypeStruct((NC, NPAD, D), jnp.float32),
    mesh=_MESH,
    scratch_types=[
        pltpu.VMEM((NB, B), jnp.int32),
        pltpu.VMEM((NB, B), jnp.int32),
        pltpu.VMEM((B, D), jnp.float32),
        pltpu.VMEM_SHARED((NPAD, D), jnp.float32),
    ],
)


def _prep_body(degt_ref, xp_ref, hs0_ref, c1_ref, c2_ref, norm_ref):
    deg = jnp.sum(degt_ref[...], axis=1, keepdims=True)
    deg = jnp.where(deg > 0, deg, 1.0)
    nrm = lax.rsqrt(deg)
    xv = xp_ref[...]
    norm_ref[...] = nrm
    c1_ref[...] = (1.0 - ALPHA) * nrm * nrm
    hs0_ref[...] = xv * nrm
    c2_ref[...] = ALPHA * nrm * xv


def _prep(degp, xp):
    full = lambda *shape: pl.BlockSpec(shape, lambda: tuple(0 for _ in shape))
    return pl.pallas_call(
        _prep_body,
        in_specs=[full(NPAD, NC), full(NPAD, D)],
        out_specs=[full(NPAD, D), full(NPAD, 1), full(NPAD, D), full(NPAD, 1)],
        out_shape=[
            jax.ShapeDtypeStruct((NPAD, D), jnp.float32),
            jax.ShapeDtypeStruct((NPAD, 1), jnp.float32),
            jax.ShapeDtypeStruct((NPAD, D), jnp.float32),
            jax.ShapeDtypeStruct((NPAD, 1), jnp.float32),
        ],
    )(degp.T, xp)


_RB = 2048


def _comb_body(aggp_ref, c1_ref, c2_ref, out_ref):
    out_ref[...] = c1_ref[...] * (aggp_ref[0] + aggp_ref[1]) + c2_ref[...]


def _combine(aggp, c1, c2):
    return pl.pallas_call(
        _comb_body,
        grid=(NPAD // _RB,),
        in_specs=[
            pl.BlockSpec((NC, _RB, D), lambda i: (0, i, 0)),
            pl.BlockSpec((_RB, 1), lambda i: (i, 0)),
            pl.BlockSpec((_RB, D), lambda i: (i, 0)),
        ],
        out_specs=pl.BlockSpec((_RB, D), lambda i: (i, 0)),
        out_shape=jax.ShapeDtypeStruct((NPAD, D), jnp.float32),
    )(aggp, c1, c2)


def _final_body(aggp_ref, xp_ref, norm_ref, w1_ref, b1_ref, w2_ref, b2_ref,
                g_ref, bt_ref, r_ref, rst_ref):
    agg = aggp_ref[0] + aggp_ref[1]
    xv = xp_ref[...]
    h = (1.0 - ALPHA) * norm_ref[...] * agg + ALPHA * xv
    r_ref[...] = h
    mu = jnp.mean(h, axis=1, keepdims=True)
    var = jnp.mean((h - mu) ** 2, axis=1, keepdims=True)
    hn = (h - mu) * lax.rsqrt(var + 1e-5) * g_ref[...] + bt_ref[...]
    inter = jnp.maximum(
        jnp.dot(hn, w1_ref[...], preferred_element_type=jnp.float32) + b1_ref[...],
        0.0,
    )
    ff = jnp.dot(inter, w2_ref[...], preferred_element_type=jnp.float32) + b2_ref[...]
    rst_ref[...] = ff + xv


def _final(aggp, xp, norm, w1, b1, w2, b2, g, bt):
    row = pl.BlockSpec((_RB, D), lambda i: (i, 0))
    mat = pl.BlockSpec((D, D), lambda i: (0, 0))
    vec = pl.BlockSpec((1, D), lambda i: (0, 0))
    return pl.pallas_call(
        _final_body,
        grid=(NPAD // _RB,),
        in_specs=[
            pl.BlockSpec((NC, _RB, D), lambda i: (0, i, 0)),
            row,
            pl.BlockSpec((_RB, 1), lambda i: (i, 0)),
            mat, vec, mat, vec, vec, vec,
        ],
        out_specs=[row, row],
        out_shape=[
            jax.ShapeDtypeStruct((NPAD, D), jnp.float32),
            jax.ShapeDtypeStruct((NPAD, D), jnp.float32),
        ],
    )(aggp, xp, norm, w1, b1.reshape(1, D), w2, b2.reshape(1, D),
      g.reshape(1, D), bt.reshape(1, D))


def kernel(x, edge_index, w1, b1, w2, b2, ln_gamma, ln_beta):
    src3 = edge_index[0].reshape(NW, NB, B)
    dst3 = edge_index[1].reshape(NW, NB, B)
    xp = jnp.pad(x, ((0, NPAD - N), (0, 0)))
    z1 = jnp.zeros((NPAD,), jnp.float32)
    z2 = jnp.zeros((NPAD, D), jnp.float32)

    degp = _deg_call(dst3, z1)
    hs, c1, c2, norm = _prep(degp, xp)
    for hop in range(HOPS):
        aggp = _hop_call(hs, src3, dst3, z2)
        if hop < HOPS - 1:
            hs = _combine(aggp, c1, c2)
    rpad, rstpad = _final(aggp, xp, norm, w1, b1, w2, b2, ln_gamma, ln_beta)
    return (rstpad[:N], rpad[:N])


# speedup vs baseline: 8.3166x; 1.1621x over previous
import functools

import jax
import jax.numpy as jnp
from jax import lax
from jax.experimental import pallas as pl
from jax.experimental.pallas import tpu as pltpu
from jax.experimental.pallas import tpu_sc as plsc

N = 10000
E = 320000
D = 128
ALPHA = 0.1
HOPS = 10

NC = 2
NS = 16
NW = NC * NS
NPAD = 10240
RPW = NPAD // NS
EPW = E // NW
B = 120
NB = 83
BT = EPW - NB * B

_MESH = plsc.VectorSubcoreMesh(
    core_axis_name="c", subcore_axis_name="s", num_cores=NC, num_subcores=NS
)


DEG_B = 80
DEG_NB = EPW // DEG_B


def _deg_body(dst3, z1, out, dst_v, ones_v, d0, deg_sh):
    c = lax.axis_index("c")
    s = lax.axis_index("s")
    w = c * NS + s
    for j in range(DEG_B // 16):
        ones_v[pl.ds(j * 16, 16)] = jnp.full((16,), 1.0, jnp.float32)
    pltpu.sync_copy(z1.at[pl.ds(s * RPW, RPW)], deg_sh.at[pl.ds(s * RPW, RPW)])
    plsc.subcore_barrier()
    pltpu.sync_copy(dst3.at[w], dst_v)

    def body(i, carry):
        pltpu.async_copy(ones_v, deg_sh.at[dst_v.at[i]], d0, add=True)
        return carry

    lax.fori_loop(0, DEG_NB, body, 0)

    def drain(i, carry):
        pltpu.make_async_copy(ones_v, deg_sh.at[dst_v.at[0]], d0).wait()
        return carry

    lax.fori_loop(0, DEG_NB, drain, 0)
    plsc.subcore_barrier()
    pltpu.sync_copy(deg_sh.at[pl.ds(s * RPW, RPW)], out.at[c, pl.ds(s * RPW, RPW)])


_deg_call = pl.kernel(
    _deg_body,
    out_type=jax.ShapeDtypeStruct((NC, NPAD), jnp.float32),
    mesh=_MESH,
    scratch_types=[
        pltpu.VMEM((DEG_NB, DEG_B), jnp.int32),
        pltpu.VMEM((DEG_B,), jnp.float32),
        pltpu.SemaphoreType.DMA,
        pltpu.VMEM_SHARED((NPAD,), jnp.float32),
    ],
)


def _hop_body(hs, srcm, dstm, srct, dstt, z2, out, src_v, dst_v, st_v, dt_v,
              rows0, rowt, g0, agg_sh):
    c = lax.axis_index("c")
    s = lax.axis_index("s")
    w = c * NS + s
    ca = pltpu.async_copy(srcm.at[w], src_v, g0)
    cb = pltpu.async_copy(dstm.at[w], dst_v, g0)
    cd = pltpu.async_copy(srct.at[w], st_v, g0)
    ce = pltpu.async_copy(dstt.at[w], dt_v, g0)
    cc = pltpu.async_copy(
        z2.at[pl.ds(s * RPW, RPW)], agg_sh.at[pl.ds(s * RPW, RPW)], g0)
    ca.wait()
    cb.wait()
    cd.wait()
    ce.wait()
    cc.wait()
    plsc.subcore_barrier()

    def body(i, carry):
        pltpu.async_copy(hs.at[src_v.at[i]], rows0, g0).wait()
        pltpu.sync_copy(rows0, agg_sh.at[dst_v.at[i]], add=True)
        return carry

    lax.fori_loop(0, NB, body, 0)
    pltpu.async_copy(hs.at[st_v], rowt, g0).wait()
    pltpu.sync_copy(rowt, agg_sh.at[dt_v], add=True)
    plsc.subcore_barrier()
    pltpu.sync_copy(agg_sh.at[pl.ds(s * RPW, RPW)], out.at[c, pl.ds(s * RPW, RPW)])


_hop_call = pl.kernel(
    _hop_body,
    out_type=jax.ShapeDtypeStruct((NC, NPAD, D), jnp.float32),
    mesh=_MESH,
    scratch_types=[
        pltpu.VMEM((NB, B), jnp.int32),
        pltpu.VMEM((NB, B), jnp.int32),
        pltpu.VMEM((BT,), jnp.int32),
        pltpu.VMEM((BT,), jnp.int32),
        pltpu.VMEM((B, D), jnp.float32),
        pltpu.VMEM((BT, D), jnp.float32),
        pltpu.SemaphoreType.DMA,
        pltpu.VMEM_SHARED((NPAD, D), jnp.float32),
    ],
)


def _prep_body(degt_ref, xp_ref, hs0_ref, c1_ref, c2_ref, norm_ref):
    deg = jnp.sum(degt_ref[...], axis=1, keepdims=True)
    deg = jnp.where(deg > 0, deg, 1.0)
    nrm = lax.rsqrt(deg)
    xv = xp_ref[...]
    norm_ref[...] = nrm
    c1_ref[...] = (1.0 - ALPHA) * nrm * nrm
    hs0_ref[...] = xv * nrm
    c2_ref[...] = ALPHA * nrm * xv


def _prep(degp, xp):
    full = lambda *shape: pl.BlockSpec(shape, lambda: tuple(0 for _ in shape))
    return pl.pallas_call(
        _prep_body,
        in_specs=[full(NPAD, NC), full(NPAD, D)],
        out_specs=[full(NPAD, D), full(NPAD, 1), full(NPAD, D), full(NPAD, 1)],
        out_shape=[
            jax.ShapeDtypeStruct((NPAD, D), jnp.float32),
            jax.ShapeDtypeStruct((NPAD, 1), jnp.float32),
            jax.ShapeDtypeStruct((NPAD, D), jnp.float32),
            jax.ShapeDtypeStruct((NPAD, 1), jnp.float32),
        ],
    )(degp.T, xp)


_RB = 2048


def _comb_body(aggp_ref, c1_ref, c2_ref, out_ref):
    out_ref[...] = c1_ref[...] * (aggp_ref[0] + aggp_ref[1]) + c2_ref[...]


def _combine(aggp, c1, c2):
    return pl.pallas_call(
        _comb_body,
        grid=(NPAD // _RB,),
        in_specs=[
            pl.BlockSpec((NC, _RB, D), lambda i: (0, i, 0)),
            pl.BlockSpec((_RB, 1), lambda i: (i, 0)),
            pl.BlockSpec((_RB, D), lambda i: (i, 0)),
        ],
        out_specs=pl.BlockSpec((_RB, D), lambda i: (i, 0)),
        out_shape=jax.ShapeDtypeStruct((NPAD, D), jnp.float32),
    )(aggp, c1, c2)


def _final_body(aggp_ref, xp_ref, norm_ref, w1_ref, b1_ref, w2_ref, b2_ref,
                g_ref, bt_ref, r_ref, rst_ref):
    agg = aggp_ref[0] + aggp_ref[1]
    xv = xp_ref[...]
    h = (1.0 - ALPHA) * norm_ref[...] * agg + ALPHA * xv
    r_ref[...] = h
    mu = jnp.mean(h, axis=1, keepdims=True)
    var = jnp.mean((h - mu) ** 2, axis=1, keepdims=True)
    hn = (h - mu) * lax.rsqrt(var + 1e-5) * g_ref[...] + bt_ref[...]
    inter = jnp.maximum(
        jnp.dot(hn, w1_ref[...], preferred_element_type=jnp.float32) + b1_ref[...],
        0.0,
    )
    ff = jnp.dot(inter, w2_ref[...], preferred_element_type=jnp.float32) + b2_ref[...]
    rst_ref[...] = ff + xv


def _final(aggp, xp, norm, w1, b1, w2, b2, g, bt):
    row = pl.BlockSpec((_RB, D), lambda i: (i, 0))
    mat = pl.BlockSpec((D, D), lambda i: (0, 0))
    vec = pl.BlockSpec((1, D), lambda i: (0, 0))
    return pl.pallas_call(
        _final_body,
        grid=(NPAD // _RB,),
        in_specs=[
            pl.BlockSpec((NC, _RB, D), lambda i: (0, i, 0)),
            row,
            pl.BlockSpec((_RB, 1), lambda i: (i, 0)),
            mat, vec, mat, vec, vec, vec,
        ],
        out_specs=[row, row],
        out_shape=[
            jax.ShapeDtypeStruct((NPAD, D), jnp.float32),
            jax.ShapeDtypeStruct((NPAD, D), jnp.float32),
        ],
    )(aggp, xp, norm, w1, b1.reshape(1, D), w2, b2.reshape(1, D),
      g.reshape(1, D), bt.reshape(1, D))


def kernel(x, edge_index, w1, b1, w2, b2, ln_gamma, ln_beta):
    src2 = edge_index[0].reshape(NW, EPW)
    dst2 = edge_index[1].reshape(NW, EPW)
    srcm = src2[:, :NB * B].reshape(NW, NB, B)
    dstm = dst2[:, :NB * B].reshape(NW, NB, B)
    srct = src2[:, NB * B:]
    dstt = dst2[:, NB * B:]
    dst3 = edge_index[1].reshape(NW, DEG_NB, DEG_B)
    xp = jnp.pad(x, ((0, NPAD - N), (0, 0)))
    z1 = jnp.zeros((NPAD,), jnp.float32)
    z2 = jnp.zeros((NPAD, D), jnp.float32)

    degp = _deg_call(dst3, z1)
    hs, c1, c2, norm = _prep(degp, xp)
    for hop in range(HOPS):
        aggp = _hop_call(hs, srcm, dstm, srct, dstt, z2)
        if hop < HOPS - 1:
            hs = _combine(aggp, c1, c2)
    rpad, rstpad = _final(aggp, xp, norm, w1, b1, w2, b2, ln_gamma, ln_beta)
    return (rstpad[:N], rpad[:N])


# speedup vs baseline: 8.3511x; 1.0041x over previous
import functools

import jax
import jax.numpy as jnp
from jax import lax
from jax.experimental import pallas as pl
from jax.experimental.pallas import tpu as pltpu
from jax.experimental.pallas import tpu_sc as plsc

N = 10000
E = 320000
D = 128
ALPHA = 0.1
HOPS = 10

NC = 2
NS = 16
NW = NC * NS
NPAD = 10240
RPW = NPAD // NS
EPW = E // NW
B = 120
NB = 83
BT = EPW - NB * B

_MESH = plsc.VectorSubcoreMesh(
    core_axis_name="c", subcore_axis_name="s", num_cores=NC, num_subcores=NS
)


DEG_B = 80
DEG_NB = EPW // DEG_B


def _deg_body(dst3, z1, out, dst_v, ones_v, d0, deg_sh):
    c = lax.axis_index("c")
    s = lax.axis_index("s")
    w = c * NS + s
    for j in range(DEG_B // 16):
        ones_v[pl.ds(j * 16, 16)] = jnp.full((16,), 1.0, jnp.float32)
    pltpu.sync_copy(z1.at[pl.ds(s * RPW, RPW)], deg_sh.at[pl.ds(s * RPW, RPW)])
    plsc.subcore_barrier()
    pltpu.sync_copy(dst3.at[w], dst_v)

    def body(i, carry):
        pltpu.async_copy(ones_v, deg_sh.at[dst_v.at[i]], d0, add=True)
        return carry

    lax.fori_loop(0, DEG_NB, body, 0)

    def drain(i, carry):
        pltpu.make_async_copy(ones_v, deg_sh.at[dst_v.at[0]], d0).wait()
        return carry

    lax.fori_loop(0, DEG_NB, drain, 0)
    plsc.subcore_barrier()
    pltpu.sync_copy(deg_sh.at[pl.ds(s * RPW, RPW)], out.at[c, pl.ds(s * RPW, RPW)])


_deg_call = pl.kernel(
    _deg_body,
    out_type=jax.ShapeDtypeStruct((NC, NPAD), jnp.float32),
    mesh=_MESH,
    scratch_types=[
        pltpu.VMEM((DEG_NB, DEG_B), jnp.int32),
        pltpu.VMEM((DEG_B,), jnp.float32),
        pltpu.SemaphoreType.DMA,
        pltpu.VMEM_SHARED((NPAD,), jnp.float32),
    ],
)


def _hop_body(hs, srcm, dstm, srct, dstt, z2, out, src_v, dst_v, st_v, dt_v,
              rows0, rowt, g0, g1, agg_sh):
    c = lax.axis_index("c")
    s = lax.axis_index("s")
    w = c * NS + s
    ca = pltpu.async_copy(srcm.at[w], src_v, g0)
    cb = pltpu.async_copy(dstm.at[w], dst_v, g0)
    cd = pltpu.async_copy(srct.at[w], st_v, g0)
    ce = pltpu.async_copy(dstt.at[w], dt_v, g0)
    cc = pltpu.async_copy(
        z2.at[pl.ds(s * RPW, RPW)], agg_sh.at[pl.ds(s * RPW, RPW)], g0)
    ca.wait()
    cb.wait()
    cd.wait()
    ce.wait()
    cc.wait()
    plsc.subcore_barrier()

    def body(i, carry):
        pltpu.async_copy(hs.at[src_v.at[i]], rows0, g0).wait()
        pltpu.sync_copy(rows0, agg_sh.at[dst_v.at[i]], add=True)
        return carry

    ct = pltpu.async_copy(hs.at[st_v], rowt, g1)
    lax.fori_loop(0, NB, body, 0)
    ct.wait()
    pltpu.sync_copy(rowt, agg_sh.at[dt_v], add=True)
    plsc.subcore_barrier()
    pltpu.sync_copy(agg_sh.at[pl.ds(s * RPW, RPW)], out.at[c, pl.ds(s * RPW, RPW)])


_hop_call = pl.kernel(
    _hop_body,
    out_type=jax.ShapeDtypeStruct((NC, NPAD, D), jnp.float32),
    mesh=_MESH,
    scratch_types=[
        pltpu.VMEM((NB, B), jnp.int32),
        pltpu.VMEM((NB, B), jnp.int32),
        pltpu.VMEM((BT,), jnp.int32),
        pltpu.VMEM((BT,), jnp.int32),
        pltpu.VMEM((B, D), jnp.float32),
        pltpu.VMEM((BT, D), jnp.float32),
        pltpu.SemaphoreType.DMA,
        pltpu.SemaphoreType.DMA,
        pltpu.VMEM_SHARED((NPAD, D), jnp.float32),
    ],
)


def _prep_body(degt_ref, xp_ref, hs0_ref, c1_ref, c2_ref, norm_ref):
    deg = jnp.sum(degt_ref[...], axis=1, keepdims=True)
    deg = jnp.where(deg > 0, deg, 1.0)
    nrm = lax.rsqrt(deg)
    xv = xp_ref[...]
    norm_ref[...] = nrm
    c1_ref[...] = (1.0 - ALPHA) * nrm * nrm
    hs0_ref[...] = xv * nrm
    c2_ref[...] = ALPHA * nrm * xv


def _prep(degp, xp):
    full = lambda *shape: pl.BlockSpec(shape, lambda: tuple(0 for _ in shape))
    return pl.pallas_call(
        _prep_body,
        in_specs=[full(NPAD, NC), full(NPAD, D)],
        out_specs=[full(NPAD, D), full(NPAD, 1), full(NPAD, D), full(NPAD, 1)],
        out_shape=[
            jax.ShapeDtypeStruct((NPAD, D), jnp.float32),
            jax.ShapeDtypeStruct((NPAD, 1), jnp.float32),
            jax.ShapeDtypeStruct((NPAD, D), jnp.float32),
            jax.ShapeDtypeStruct((NPAD, 1), jnp.float32),
        ],
    )(degp.T, xp)


_RB = 2048


def _comb_body(aggp_ref, c1_ref, c2_ref, out_ref):
    out_ref[...] = c1_ref[...] * (aggp_ref[0] + aggp_ref[1]) + c2_ref[...]


def _combine(aggp, c1, c2):
    return pl.pallas_call(
        _comb_body,
        grid=(NPAD // _RB,),
        in_specs=[
            pl.BlockSpec((NC, _RB, D), lambda i: (0, i, 0)),
            pl.BlockSpec((_RB, 1), lambda i: (i, 0)),
            pl.BlockSpec((_RB, D), lambda i: (i, 0)),
        ],
        out_specs=pl.BlockSpec((_RB, D), lambda i: (i, 0)),
        out_shape=jax.ShapeDtypeStruct((NPAD, D), jnp.float32),
    )(aggp, c1, c2)


def _final_body(aggp_ref, xp_ref, norm_ref, w1_ref, b1_ref, w2_ref, b2_ref,
                g_ref, bt_ref, r_ref, rst_ref):
    agg = aggp_ref[0] + aggp_ref[1]
    xv = xp_ref[...]
    h = (1.0 - ALPHA) * norm_ref[...] * agg + ALPHA * xv
    r_ref[...] = h
    mu = jnp.mean(h, axis=1, keepdims=True)
    var = jnp.mean((h - mu) ** 2, axis=1, keepdims=True)
    hn = (h - mu) * lax.rsqrt(var + 1e-5) * g_ref[...] + bt_ref[...]
    inter = jnp.maximum(
        jnp.dot(hn, w1_ref[...], preferred_element_type=jnp.float32) + b1_ref[...],
        0.0,
    )
    ff = jnp.dot(inter, w2_ref[...], preferred_element_type=jnp.float32) + b2_ref[...]
    rst_ref[...] = ff + xv


def _final(aggp, xp, norm, w1, b1, w2, b2, g, bt):
    row = pl.BlockSpec((_RB, D), lambda i: (i, 0))
    mat = pl.BlockSpec((D, D), lambda i: (0, 0))
    vec = pl.BlockSpec((1, D), lambda i: (0, 0))
    return pl.pallas_call(
        _final_body,
        grid=(NPAD // _RB,),
        in_specs=[
            pl.BlockSpec((NC, _RB, D), lambda i: (0, i, 0)),
            row,
            pl.BlockSpec((_RB, 1), lambda i: (i, 0)),
            mat, vec, mat, vec, vec, vec,
        ],
        out_specs=[row, row],
        out_shape=[
            jax.ShapeDtypeStruct((NPAD, D), jnp.float32),
            jax.ShapeDtypeStruct((NPAD, D), jnp.float32),
        ],
    )(aggp, xp, norm, w1, b1.reshape(1, D), w2, b2.reshape(1, D),
      g.reshape(1, D), bt.reshape(1, D))


def kernel(x, edge_index, w1, b1, w2, b2, ln_gamma, ln_beta):
    src2 = edge_index[0].reshape(NW, EPW)
    dst2 = edge_index[1].reshape(NW, EPW)
    srcm = src2[:, :NB * B].reshape(NW, NB, B)
    dstm = dst2[:, :NB * B].reshape(NW, NB, B)
    srct = src2[:, NB * B:]
    dstt = dst2[:, NB * B:]
    dst3 = edge_index[1].reshape(NW, DEG_NB, DEG_B)
    xp = jnp.pad(x, ((0, NPAD - N), (0, 0)))
    z1 = jnp.zeros((NPAD,), jnp.float32)
    z2 = jnp.zeros((NPAD, D), jnp.float32)

    degp = _deg_call(dst3, z1)
    hs, c1, c2, norm = _prep(degp, xp)
    for hop in range(HOPS):
        aggp = _hop_call(hs, srcm, dstm, srct, dstt, z2)
        if hop < HOPS - 1:
            hs = _combine(aggp, c1, c2)
    rpad, rstpad = _final(aggp, xp, norm, w1, b1, w2, b2, ln_gamma, ln_beta)
    return (rstpad[:N], rpad[:N])
